# baseline (speedup 1.0000x reference)
import functools

import jax
import jax.numpy as jnp
from jax import lax
from jax.experimental import pallas as pl
from jax.experimental.pallas import tpu as pltpu
from jax.experimental.pallas import tpu_sc as plsc

_NUM_CODES = 8192
_CODE_DIM = 256
_TOKENS = 16384
_HALF = _TOKENS // 2
_T_BLK = 1024
_C_BLK = 2048
_N_CHUNKS = _NUM_CODES // _C_BLK


def _argmin_body(x_ref, e_ref, l2e_ref, codes_ref):
    x = x_ref[...]
    xm2 = -2.0 * x
    l2x = jnp.sum(x * x, axis=1, keepdims=True)

    best_val = jnp.full((_T_BLK, 1), jnp.inf, dtype=jnp.float32)
    best_idx = jnp.full((_T_BLK, 1), 0.0, dtype=jnp.float32)

    for c in range(_N_CHUNKS):
        ec = e_ref[pl.ds(c * _C_BLK, _C_BLK), :]
        l2e = l2e_ref[:, pl.ds(c * _C_BLK, _C_BLK)]
        nd2 = lax.dot_general(
            xm2, ec, (((1,), (1,)), ((), ())),
            preferred_element_type=jnp.float32)
        dist = (l2x + l2e) + nd2
        m = jnp.min(dist, axis=1, keepdims=True)
        iota = lax.broadcasted_iota(
            jnp.int32, (_T_BLK, _C_BLK), 1).astype(jnp.float32)
        idx = jnp.min(
            jnp.where(dist <= m, iota, float(_C_BLK)),
            axis=1, keepdims=True)
        take = m < best_val
        best_val = jnp.where(take, m, best_val)
        best_idx = jnp.where(take, idx + float(c * _C_BLK), best_idx)

    codes_ref[...] = best_idx.astype(jnp.int32)


def _tc_codes(x, e, l2e):
    n_tok = x.shape[0]
    return pl.pallas_call(
        _argmin_body,
        grid=(n_tok // _T_BLK,),
        in_specs=[
            pl.BlockSpec((_T_BLK, _CODE_DIM), lambda i: (i, 0)),
            pl.BlockSpec((_NUM_CODES, _CODE_DIM), lambda i: (0, 0)),
            pl.BlockSpec((1, _NUM_CODES), lambda i: (0, 0)),
        ],
        out_specs=pl.BlockSpec((_T_BLK, 1), lambda i: (i, 0)),
        out_shape=jax.ShapeDtypeStruct((n_tok, 1), jnp.int32),
    )(x, e, l2e)


_NW = 32
_G_CHUNK = 128


@functools.cache
def _sc_gather_kernel(n_rows):
    rows_per_w = n_rows // _NW

    @functools.partial(
        pl.kernel,
        out_type=jax.ShapeDtypeStruct((n_rows, _CODE_DIM), jnp.float32),
        mesh=plsc.VectorSubcoreMesh(
            core_axis_name="c", subcore_axis_name="s",
            num_cores=2, num_subcores=16),
        scratch_types=[
            pltpu.VMEM((rows_per_w,), jnp.int32),
            pltpu.VMEM((_G_CHUNK, _CODE_DIM), jnp.float32),
            pltpu.VMEM((_G_CHUNK, _CODE_DIM), jnp.float32),
            pltpu.SemaphoreType.DMA,
            pltpu.SemaphoreType.DMA,
            pltpu.SemaphoreType.DMA,
            pltpu.SemaphoreType.DMA,
        ],
    )
    def _sc_gather(table_hbm, idx_hbm, out_hbm,
                   idx_all, rows0, rows1, gs0, gs1, os0, os1):
        wid = lax.axis_index("s") * 2 + lax.axis_index("c")
        base = wid * rows_per_w
        n = rows_per_w // _G_CHUNK
        rows, gs, os = [rows0, rows1], [gs0, gs1], [os0, os1]
        pltpu.sync_copy(idx_hbm.at[pl.ds(base, rows_per_w)], idx_all)
        gh = [
            pltpu.async_copy(
                table_hbm.at[idx_all.at[pl.ds(g * _G_CHUNK, _G_CHUNK)]],
                rows[g], gs[g])
            for g in range(min(2, n))
        ]
        oh = [None, None]
        for g in range(n):
            b = g % 2
            gh[b].wait()
            oh[b] = pltpu.async_copy(
                rows[b], out_hbm.at[pl.ds(base + g * _G_CHUNK, _G_CHUNK)],
                os[b])
            if g + 2 < n:
                oh[b].wait()
                gh[b] = pltpu.async_copy(
                    table_hbm.at[
                        idx_all.at[pl.ds((g + 2) * _G_CHUNK, _G_CHUNK)]],
                    rows[b], gs[b])
        oh[0].wait()
        if n > 1:
            oh[1].wait()

    return _sc_gather


def kernel(inputs, embeddings):
    x = inputs.reshape(_TOKENS, _CODE_DIM)
    l2e = jnp.sum(embeddings ** 2, axis=-1).reshape(1, _NUM_CODES)
    gather = _sc_gather_kernel(_HALF)
    codes_a = _tc_codes(x[:_HALF], embeddings, l2e)
    out_a = gather(embeddings, codes_a.reshape(_HALF))
    codes_b = _tc_codes(x[_HALF:], embeddings, l2e)
    out_b = gather(embeddings, codes_b.reshape(_HALF))
    out = jnp.concatenate([out_a, out_b], axis=0)
    return out.reshape(inputs.shape)


# speedup vs baseline: 1.1644x; 1.1644x over previous
import functools

import jax
import jax.numpy as jnp
from jax import lax
from jax.experimental import pallas as pl
from jax.experimental.pallas import tpu as pltpu
from jax.experimental.pallas import tpu_sc as plsc

_NUM_CODES = 8192
_CODE_DIM = 256
_TOKENS = 16384
_HALF = _TOKENS // 2
_T_BLK = 1024
_C_BLK = 2048
_N_CHUNKS = _NUM_CODES // _C_BLK


def _argmin_body(x_ref, e_ref, l2e_ref, codes_ref):
    x = x_ref[...]
    xm2 = -2.0 * x
    l2x = jnp.sum(x * x, axis=1, keepdims=True)

    best_val = jnp.full((_T_BLK, 1), jnp.inf, dtype=jnp.float32)
    best_idx = jnp.full((_T_BLK, 1), 0.0, dtype=jnp.float32)

    for c in range(_N_CHUNKS):
        ec = e_ref[pl.ds(c * _C_BLK, _C_BLK), :]
        l2e = l2e_ref[:, pl.ds(c * _C_BLK, _C_BLK)]
        nd2 = lax.dot_general(
            xm2, ec, (((1,), (1,)), ((), ())),
            preferred_element_type=jnp.float32)
        dist = (l2x + l2e) + nd2
        m = jnp.min(dist, axis=1, keepdims=True)
        iota = lax.broadcasted_iota(
            jnp.int32, (_T_BLK, _C_BLK), 1).astype(jnp.float32)
        idx = jnp.min(
            jnp.where(dist <= m, iota, float(_C_BLK)),
            axis=1, keepdims=True)
        take = m < best_val
        best_val = jnp.where(take, m, best_val)
        best_idx = jnp.where(take, idx + float(c * _C_BLK), best_idx)

    codes_ref[...] = best_idx.astype(jnp.int32)


def _tc_codes(x, e, l2e):
    n_tok = x.shape[0]
    return pl.pallas_call(
        _argmin_body,
        grid=(n_tok // _T_BLK,),
        in_specs=[
            pl.BlockSpec((_T_BLK, _CODE_DIM), lambda i: (i, 0)),
            pl.BlockSpec((_NUM_CODES, _CODE_DIM), lambda i: (0, 0)),
            pl.BlockSpec((1, _NUM_CODES), lambda i: (0, 0)),
        ],
        out_specs=pl.BlockSpec((_T_BLK, 1), lambda i: (i, 0)),
        out_shape=jax.ShapeDtypeStruct((n_tok, 1), jnp.int32),
    )(x, e, l2e)


_NW = 32
_G_CHUNK = 128


@functools.cache
def _sc_gather_kernel(n_rows):
    rows_per_w = n_rows // _NW

    @functools.partial(
        pl.kernel,
        out_type=jax.ShapeDtypeStruct((n_rows, _CODE_DIM), jnp.float32),
        mesh=plsc.VectorSubcoreMesh(
            core_axis_name="c", subcore_axis_name="s",
            num_cores=2, num_subcores=16),
        scratch_types=[
            pltpu.VMEM((rows_per_w,), jnp.int32),
            pltpu.VMEM((_G_CHUNK, _CODE_DIM), jnp.float32),
            pltpu.VMEM((_G_CHUNK, _CODE_DIM), jnp.float32),
            pltpu.SemaphoreType.DMA,
            pltpu.SemaphoreType.DMA,
            pltpu.SemaphoreType.DMA,
            pltpu.SemaphoreType.DMA,
        ],
    )
    def _sc_gather(table_hbm, idx_hbm, out_hbm,
                   idx_all, rows0, rows1, gs0, gs1, os0, os1):
        wid = lax.axis_index("s") * 2 + lax.axis_index("c")
        base = wid * rows_per_w
        n = rows_per_w // _G_CHUNK
        rows, gs, os = [rows0, rows1], [gs0, gs1], [os0, os1]
        pltpu.sync_copy(idx_hbm.at[pl.ds(base, rows_per_w)], idx_all)
        gh = [
            pltpu.async_copy(
                table_hbm.at[idx_all.at[pl.ds(g * _G_CHUNK, _G_CHUNK)]],
                rows[g], gs[g])
            for g in range(min(2, n))
        ]
        oh = [None, None]
        for g in range(n):
            b = g % 2
            gh[b].wait()
            oh[b] = pltpu.async_copy(
                rows[b], out_hbm.at[pl.ds(base + g * _G_CHUNK, _G_CHUNK)],
                os[b])
            if g + 2 < n:
                oh[b].wait()
                gh[b] = pltpu.async_copy(
                    table_hbm.at[
                        idx_all.at[pl.ds((g + 2) * _G_CHUNK, _G_CHUNK)]],
                    rows[b], gs[b])
        oh[0].wait()
        if n > 1:
            oh[1].wait()

    return _sc_gather


def kernel(inputs, embeddings):
    x = inputs.reshape(_TOKENS, _CODE_DIM)
    l2e = jnp.sum(embeddings ** 2, axis=-1).reshape(1, _NUM_CODES)
    gather = _sc_gather_kernel(_TOKENS)
    codes = _tc_codes(x, embeddings, l2e)
    out = gather(embeddings, codes.reshape(_TOKENS))
    return out.reshape(inputs.shape)


# speedup vs baseline: 1.3348x; 1.1463x over previous
import functools

import jax
import jax.numpy as jnp
from jax import lax
from jax.experimental import pallas as pl
from jax.experimental.pallas import tpu as pltpu
from jax.experimental.pallas import tpu_sc as plsc

_NUM_CODES = 8192
_CODE_DIM = 256
_TOKENS = 16384
_HALF = _TOKENS // 2
_T_BLK = 256
_C_BLK = 2048
_N_CHUNKS = _NUM_CODES // _C_BLK
_S_BLK = 128
_N_SLICES = _C_BLK // _S_BLK


def _argmin_body(x_ref, e_ref, l2e_ref, codes_ref):
    x = x_ref[...]
    xm2 = -2.0 * x
    l2x = jnp.sum(x * x, axis=1, keepdims=True)

    best_val = jnp.full((_T_BLK, _S_BLK), jnp.inf, dtype=jnp.float32)
    best_g = jnp.zeros((_T_BLK, _S_BLK), dtype=jnp.float32)

    for c in range(_N_CHUNKS):
        ec = e_ref[pl.ds(c * _C_BLK, _C_BLK), :]
        nd2 = lax.dot_general(
            xm2, ec, (((1,), (1,)), ((), ())),
            preferred_element_type=jnp.float32)
        for s in range(_N_SLICES):
            g = c * _N_SLICES + s
            l2e = l2e_ref[:, pl.ds(g * _S_BLK, _S_BLK)]
            dist = (l2x + l2e) + nd2[:, s * _S_BLK:(s + 1) * _S_BLK]
            take = dist < best_val
            best_val = jnp.where(take, dist, best_val)
            best_g = jnp.where(take, jnp.float32(g), best_g)

    lane = lax.broadcasted_iota(
        jnp.int32, (_T_BLK, _S_BLK), 1).astype(jnp.float32)
    full_idx = best_g * jnp.float32(_S_BLK) + lane
    m = jnp.min(best_val, axis=1, keepdims=True)
    idx = jnp.min(
        jnp.where(best_val <= m, full_idx, jnp.float32(_NUM_CODES)),
        axis=1, keepdims=True)
    codes_ref[...] = idx.astype(jnp.int32)


def _tc_codes(x, e, l2e):
    n_tok = x.shape[0]
    return pl.pallas_call(
        _argmin_body,
        grid=(n_tok // _T_BLK,),
        in_specs=[
            pl.BlockSpec((_T_BLK, _CODE_DIM), lambda i: (i, 0)),
            pl.BlockSpec((_NUM_CODES, _CODE_DIM), lambda i: (0, 0)),
            pl.BlockSpec((1, _NUM_CODES), lambda i: (0, 0)),
        ],
        out_specs=pl.BlockSpec((_T_BLK, 1), lambda i: (i, 0)),
        out_shape=jax.ShapeDtypeStruct((n_tok, 1), jnp.int32),
    )(x, e, l2e)


_NW = 32
_G_CHUNK = 128


@functools.cache
def _sc_gather_kernel(n_rows):
    rows_per_w = n_rows // _NW

    @functools.partial(
        pl.kernel,
        out_type=jax.ShapeDtypeStruct((n_rows, _CODE_DIM), jnp.float32),
        mesh=plsc.VectorSubcoreMesh(
            core_axis_name="c", subcore_axis_name="s",
            num_cores=2, num_subcores=16),
        scratch_types=[
            pltpu.VMEM((rows_per_w,), jnp.int32),
            pltpu.VMEM((_G_CHUNK, _CODE_DIM), jnp.float32),
            pltpu.VMEM((_G_CHUNK, _CODE_DIM), jnp.float32),
            pltpu.SemaphoreType.DMA,
            pltpu.SemaphoreType.DMA,
            pltpu.SemaphoreType.DMA,
            pltpu.SemaphoreType.DMA,
        ],
    )
    def _sc_gather(table_hbm, idx_hbm, out_hbm,
                   idx_all, rows0, rows1, gs0, gs1, os0, os1):
        wid = lax.axis_index("s") * 2 + lax.axis_index("c")
        base = wid * rows_per_w
        n = rows_per_w // _G_CHUNK
        rows, gs, os = [rows0, rows1], [gs0, gs1], [os0, os1]
        pltpu.sync_copy(idx_hbm.at[pl.ds(base, rows_per_w)], idx_all)
        gh = [
            pltpu.async_copy(
                table_hbm.at[idx_all.at[pl.ds(g * _G_CHUNK, _G_CHUNK)]],
                rows[g], gs[g])
            for g in range(min(2, n))
        ]
        oh = [None, None]
        for g in range(n):
            b = g % 2
            gh[b].wait()
            oh[b] = pltpu.async_copy(
                rows[b], out_hbm.at[pl.ds(base + g * _G_CHUNK, _G_CHUNK)],
                os[b])
            if g + 2 < n:
                oh[b].wait()
                gh[b] = pltpu.async_copy(
                    table_hbm.at[
                        idx_all.at[pl.ds((g + 2) * _G_CHUNK, _G_CHUNK)]],
                    rows[b], gs[b])
        oh[0].wait()
        if n > 1:
            oh[1].wait()

    return _sc_gather


def kernel(inputs, embeddings):
    x = inputs.reshape(_TOKENS, _CODE_DIM)
    l2e = jnp.sum(embeddings ** 2, axis=-1).reshape(1, _NUM_CODES)
    gather = _sc_gather_kernel(_TOKENS)
    codes = _tc_codes(x, embeddings, l2e)
    out = gather(embeddings, codes.reshape(_TOKENS))
    return out.reshape(inputs.shape)
